# baseline (speedup 1.0000x reference)
import functools

import jax
import jax.numpy as jnp
from jax import lax
from jax.experimental import pallas as pl
from jax.experimental.pallas import tpu as pltpu
from jax.experimental.pallas import tpu_sc as plsc

EPS = 1e-05

NC = 2
NS = 16
L = 16
NW = NC * NS

_f32 = jnp.float32
_i32 = jnp.int32


def _degree_kernel(n_pad, n_batches, batch):
    hist_len = 2 * n_pad
    seg = hist_len // NS

    mesh = plsc.VectorSubcoreMesh(core_axis_name="c", subcore_axis_name="s")

    @functools.partial(
        pl.kernel,
        out_type=jax.ShapeDtypeStruct((NC * hist_len,), _f32),
        mesh=mesh,
        scratch_types=[
            pltpu.VMEM_SHARED((hist_len,), _f32),
            pltpu.VMEM((n_batches, batch), _i32),
            pltpu.VMEM((batch,), _f32),
        ],
    )
    def deg_k(hidx_hbm, zeros_hbm, out_hbm, hist_sh, idx_v, ones_v):
        c = lax.axis_index("c")
        s = lax.axis_index("s")
        w = c * NS + s
        soff = pl.multiple_of(s * seg, 128)
        ooff = pl.multiple_of(c * hist_len + s * seg, 128)

        for k in range(batch // L):
            ones_v[pl.ds(k * L, L)] = jnp.ones((L,), _f32)

        pltpu.sync_copy(zeros_hbm, hist_sh.at[pl.ds(soff, seg)])
        plsc.subcore_barrier()

        pltpu.sync_copy(hidx_hbm.at[w], idx_v)

        def body(j, _):
            pltpu.sync_copy(ones_v, hist_sh.at[idx_v.at[j]], add=True)
            return _

        lax.fori_loop(0, n_batches, body, None)
        plsc.subcore_barrier()

        pltpu.sync_copy(hist_sh.at[pl.ds(soff, seg)],
                        out_hbm.at[pl.ds(ooff, seg)])

    return deg_k


def _agg_kernel(n_pad, n_batches, batch, dtype):
    rows_seg = n_pad // NS

    mesh = plsc.VectorSubcoreMesh(core_axis_name="c", subcore_axis_name="s")

    chunk = 8
    nq = n_batches // chunk

    @functools.partial(
        pl.kernel,
        out_type=jax.ShapeDtypeStruct((NC, n_pad, 128), dtype),
        mesh=mesh,
        scratch_types=[
            pltpu.VMEM_SHARED((n_pad, 128), dtype),
            pltpu.VMEM((2, chunk, batch), _i32),
            pltpu.VMEM((2, chunk, batch), _i32),
            pltpu.VMEM((2, batch, 128), dtype),
        ] + [pltpu.SemaphoreType.DMA] * 6,
    )
    def agg_k(h_hbm, src_hbm, dst_hbm, zrows_hbm, out_hbm,
              agg_sh, sidx_c, didx_c, rows_v, *sems):
        gsems = sems[0:2]
        ssems = sems[2:4]
        isem_s, isem_d = sems[4], sems[5]
        c = lax.axis_index("c")
        s = lax.axis_index("s")
        w = c * NS + s
        roff = pl.multiple_of(s * rows_seg, 8)

        pltpu.sync_copy(zrows_hbm, agg_sh.at[pl.ds(roff, rows_seg)])
        pltpu.sync_copy(src_hbm.at[w, pl.ds(0, chunk)], sidx_c.at[0])
        pltpu.sync_copy(dst_hbm.at[w, pl.ds(0, chunk)], didx_c.at[0])
        plsc.subcore_barrier()

        pltpu.async_copy(h_hbm.at[sidx_c.at[0, 0]], rows_v.at[0], gsems[0])

        @pl.loop(0, nq)
        def _(q):
            qm = lax.rem(q, 2)
            qn = lax.rem(q + 1, 2)
            qoff = pl.multiple_of((q + 1) * chunk, chunk)

            @pl.when(q < nq - 1)
            def _pf():
                pltpu.async_copy(src_hbm.at[w, pl.ds(qoff, chunk)],
                                 sidx_c.at[qn], isem_s)
                pltpu.async_copy(dst_hbm.at[w, pl.ds(qoff, chunk)],
                                 didx_c.at[qn], isem_d)

            for jj in range(chunk):
                b = jj % 2
                nb = 1 - b
                pltpu.make_async_copy(
                    h_hbm.at[sidx_c.at[qm, jj]], rows_v.at[b],
                    gsems[b]).wait()
                pltpu.async_copy(rows_v.at[b], agg_sh.at[didx_c.at[qm, jj]],
                                 ssems[b], add=True)
                if jj == 0:
                    @pl.when(q > 0)
                    def _w0():
                        pltpu.make_async_copy(
                            rows_v.at[nb], agg_sh.at[didx_c.at[qm, jj]],
                            ssems[nb]).wait()
                    pltpu.async_copy(h_hbm.at[sidx_c.at[qm, jj + 1]],
                                     rows_v.at[nb], gsems[nb])
                elif jj < chunk - 1:
                    pltpu.make_async_copy(
                        rows_v.at[nb], agg_sh.at[didx_c.at[qm, jj]],
                        ssems[nb]).wait()
                    pltpu.async_copy(h_hbm.at[sidx_c.at[qm, jj + 1]],
                                     rows_v.at[nb], gsems[nb])
                else:
                    @pl.when(q < nq - 1)
                    def _gnext():
                        pltpu.make_async_copy(
                            src_hbm.at[w, pl.ds(qoff, chunk)],
                            sidx_c.at[qn], isem_s).wait()
                        pltpu.make_async_copy(
                            dst_hbm.at[w, pl.ds(qoff, chunk)],
                            didx_c.at[qn], isem_d).wait()
                        pltpu.make_async_copy(
                            rows_v.at[nb], agg_sh.at[didx_c.at[qm, jj]],
                            ssems[nb]).wait()
                        pltpu.async_copy(h_hbm.at[sidx_c.at[qn, 0]],
                                         rows_v.at[nb], gsems[nb])

        pltpu.make_async_copy(
            rows_v.at[0], agg_sh.at[didx_c.at[0, 0]], ssems[0]).wait()
        pltpu.make_async_copy(
            rows_v.at[1], agg_sh.at[didx_c.at[0, 0]], ssems[1]).wait()
        plsc.subcore_barrier()

        pltpu.sync_copy(agg_sh.at[pl.ds(roff, rows_seg)],
                        out_hbm.at[c, pl.ds(roff, rows_seg)])

    return agg_k


def _mm_body(x_ref, w_ref, degs_ref, h_ref, nd_ref):
    degs = degs_ref[...]
    ds = degs[0, 0] + degs[1, 0]
    dd = degs[0, 1] + degs[1, 1]
    ns = jnp.where(ds > 0, lax.rsqrt(jnp.maximum(ds, 1.0)), 0.0)
    nd = jnp.where(dd > 0, lax.rsqrt(jnp.maximum(dd, 1.0)), 0.0)
    h = jnp.dot(x_ref[...], w_ref[...], preferred_element_type=_f32)
    h_ref[...] = (h * ns[:, None]).astype(h_ref.dtype)
    nd_ref[...] = nd[:, None]


def _ep_body(parts_ref, nd_ref, b_ref, x_ref, o_ref):
    parts = parts_ref[...].astype(_f32)
    agg = parts[0] + parts[1]
    agg = agg * nd_ref[...] + b_ref[...]
    mean = jnp.mean(agg, axis=1, keepdims=True)
    cen = agg - mean
    var = jnp.mean(cen * cen, axis=1, keepdims=True)
    hn = cen / jnp.sqrt(var + EPS)
    o_ref[...] = jnp.maximum(hn, 0.0) + x_ref[...]


def kernel(x, edge_index, W, b):
    n, d_in = x.shape
    d_out = W.shape[1]
    e = edge_index.shape[1]

    n_pad = ((n + 1 + 2047) // 2048) * 2048
    batch = 128
    ept = -(-e // NW)
    n_batches = -(-(-(-ept // batch)) // 8) * 8
    e_pad = NW * n_batches * batch

    src = edge_index[0].astype(_i32)
    dst = edge_index[1].astype(_i32)
    pad_i = n + jnp.arange(e_pad - e, dtype=_i32) % (n_pad - n)
    pad_s = pad_i
    pad_d = pad_i
    src_p = jnp.concatenate([src, pad_s]).reshape(NW, n_batches, batch)
    dst_p = jnp.concatenate([dst, pad_d]).reshape(NW, n_batches, batch)
    hidx = jnp.concatenate([src_p, dst_p + n_pad], axis=1)

    x_pad = jnp.pad(x, ((0, n_pad - n), (0, 0)))

    deg_k = _degree_kernel(n_pad, 2 * n_batches, batch)
    zeros_h = jnp.zeros((2 * n_pad // NS,), _f32)
    degs = deg_k(hidx, zeros_h).reshape(NC, 2, n_pad)


    blk = 1024
    grid_mm = n_pad // blk
    h_scaled, norm_dst = pl.pallas_call(
        _mm_body,
        grid=(grid_mm,),
        in_specs=[
            pl.BlockSpec((blk, d_in), lambda i: (i, 0)),
            pl.BlockSpec((d_in, d_out), lambda i: (0, 0)),
            pl.BlockSpec((NC, 2, blk), lambda i: (0, 0, i)),
        ],
        out_specs=[
            pl.BlockSpec((blk, d_out), lambda i: (i, 0)),
            pl.BlockSpec((blk, 1), lambda i: (i, 0)),
        ],
        out_shape=[
            jax.ShapeDtypeStruct((n_pad, d_out), _f32),
            jax.ShapeDtypeStruct((n_pad, 1), _f32),
        ],
    )(x_pad, W, degs)

    agg_k = _agg_kernel(n_pad, n_batches, batch, _f32)
    zrows = jnp.zeros((n_pad // NS, 128), _f32)
    parts = agg_k(h_scaled, src_p, dst_p, zrows)

    eblk = 2000
    out = pl.pallas_call(
        _ep_body,
        grid=(n // eblk,),
        in_specs=[
            pl.BlockSpec((NC, eblk, 128), lambda i: (0, i, 0)),
            pl.BlockSpec((eblk, 1), lambda i: (i, 0)),
            pl.BlockSpec((1, 128), lambda i: (0, 0)),
            pl.BlockSpec((eblk, 128), lambda i: (i, 0)),
        ],
        out_specs=pl.BlockSpec((eblk, 128), lambda i: (i, 0)),
        out_shape=jax.ShapeDtypeStruct((n, d_out), _f32),
    )(parts, norm_dst[:n], b.reshape(1, 128), x)

    return out


# speedup vs baseline: 1.0055x; 1.0055x over previous
import functools

import jax
import jax.numpy as jnp
from jax import lax
from jax.experimental import pallas as pl
from jax.experimental.pallas import tpu as pltpu
from jax.experimental.pallas import tpu_sc as plsc

EPS = 1e-05

NC = 2
NS = 16
L = 16
NW = NC * NS

_f32 = jnp.float32
_i32 = jnp.int32


def _degree_kernel(n_pad, n_batches, batch):
    seg = n_pad // NS

    mesh = plsc.VectorSubcoreMesh(core_axis_name="c", subcore_axis_name="s")

    @functools.partial(
        pl.kernel,
        out_type=jax.ShapeDtypeStruct((NC * 2 * n_pad,), _f32),
        mesh=mesh,
        scratch_types=[
            pltpu.VMEM_SHARED((n_pad,), _f32),
            pltpu.VMEM_SHARED((n_pad,), _f32),
            pltpu.VMEM((n_batches, batch), _i32),
            pltpu.VMEM((batch,), _f32),
        ],
    )
    def deg_k(src_hbm, dst_hbm, zeros_hbm, out_hbm,
              hist_s, hist_d, idx_v, ones_v):
        c = lax.axis_index("c")
        s = lax.axis_index("s")
        w = c * NS + s
        soff = pl.multiple_of(s * seg, 128)
        ooff_s = pl.multiple_of(c * 2 * n_pad + s * seg, 128)
        ooff_d = pl.multiple_of(c * 2 * n_pad + n_pad + s * seg, 128)

        for k in range(batch // L):
            ones_v[pl.ds(k * L, L)] = jnp.ones((L,), _f32)

        pltpu.sync_copy(zeros_hbm, hist_s.at[pl.ds(soff, seg)])
        pltpu.sync_copy(zeros_hbm, hist_d.at[pl.ds(soff, seg)])
        plsc.subcore_barrier()

        pltpu.sync_copy(src_hbm.at[w], idx_v)

        def body_s(j, _):
            pltpu.sync_copy(ones_v, hist_s.at[idx_v.at[j]], add=True)
            return _

        lax.fori_loop(0, n_batches, body_s, None)
        pltpu.sync_copy(dst_hbm.at[w], idx_v)

        def body_d(j, _):
            pltpu.sync_copy(ones_v, hist_d.at[idx_v.at[j]], add=True)
            return _

        lax.fori_loop(0, n_batches, body_d, None)
        plsc.subcore_barrier()

        pltpu.sync_copy(hist_s.at[pl.ds(soff, seg)],
                        out_hbm.at[pl.ds(ooff_s, seg)])
        pltpu.sync_copy(hist_d.at[pl.ds(soff, seg)],
                        out_hbm.at[pl.ds(ooff_d, seg)])

    return deg_k


def _agg_kernel(n_pad, n_batches, batch, dtype):
    rows_seg = n_pad // NS

    mesh = plsc.VectorSubcoreMesh(core_axis_name="c", subcore_axis_name="s")

    chunk = 8
    nq = n_batches // chunk

    @functools.partial(
        pl.kernel,
        out_type=jax.ShapeDtypeStruct((NC, n_pad, 128), dtype),
        mesh=mesh,
        scratch_types=[
            pltpu.VMEM_SHARED((n_pad, 128), dtype),
            pltpu.VMEM((2, chunk, batch), _i32),
            pltpu.VMEM((2, chunk, batch), _i32),
            pltpu.VMEM((2, batch, 128), dtype),
        ] + [pltpu.SemaphoreType.DMA] * 6,
    )
    def agg_k(h_hbm, src_hbm, dst_hbm, zrows_hbm, out_hbm,
              agg_sh, sidx_c, didx_c, rows_v, *sems):
        gsems = sems[0:2]
        ssems = sems[2:4]
        isem_s, isem_d = sems[4], sems[5]
        c = lax.axis_index("c")
        s = lax.axis_index("s")
        w = c * NS + s
        roff = pl.multiple_of(s * rows_seg, 8)

        pltpu.sync_copy(zrows_hbm, agg_sh.at[pl.ds(roff, rows_seg)])
        pltpu.sync_copy(src_hbm.at[w, pl.ds(0, chunk)], sidx_c.at[0])
        pltpu.sync_copy(dst_hbm.at[w, pl.ds(0, chunk)], didx_c.at[0])
        plsc.subcore_barrier()

        pltpu.async_copy(h_hbm.at[sidx_c.at[0, 0]], rows_v.at[0], gsems[0])

        @pl.loop(0, nq)
        def _(q):
            qm = lax.rem(q, 2)
            qn = lax.rem(q + 1, 2)
            qoff = pl.multiple_of((q + 1) * chunk, chunk)

            @pl.when(q < nq - 1)
            def _pf():
                pltpu.async_copy(src_hbm.at[w, pl.ds(qoff, chunk)],
                                 sidx_c.at[qn], isem_s)
                pltpu.async_copy(dst_hbm.at[w, pl.ds(qoff, chunk)],
                                 didx_c.at[qn], isem_d)

            for jj in range(chunk):
                b = jj % 2
                nb = 1 - b
                pltpu.make_async_copy(
                    h_hbm.at[sidx_c.at[qm, jj]], rows_v.at[b],
                    gsems[b]).wait()
                pltpu.async_copy(rows_v.at[b], agg_sh.at[didx_c.at[qm, jj]],
                                 ssems[b], add=True)
                if jj == 0:
                    @pl.when(q > 0)
                    def _w0():
                        pltpu.make_async_copy(
                            rows_v.at[nb], agg_sh.at[didx_c.at[qm, jj]],
                            ssems[nb]).wait()
                    pltpu.async_copy(h_hbm.at[sidx_c.at[qm, jj + 1]],
                                     rows_v.at[nb], gsems[nb])
                elif jj < chunk - 1:
                    pltpu.make_async_copy(
                        rows_v.at[nb], agg_sh.at[didx_c.at[qm, jj]],
                        ssems[nb]).wait()
                    pltpu.async_copy(h_hbm.at[sidx_c.at[qm, jj + 1]],
                                     rows_v.at[nb], gsems[nb])
                else:
                    @pl.when(q < nq - 1)
                    def _gnext():
                        pltpu.make_async_copy(
                            src_hbm.at[w, pl.ds(qoff, chunk)],
                            sidx_c.at[qn], isem_s).wait()
                        pltpu.make_async_copy(
                            dst_hbm.at[w, pl.ds(qoff, chunk)],
                            didx_c.at[qn], isem_d).wait()
                        pltpu.make_async_copy(
                            rows_v.at[nb], agg_sh.at[didx_c.at[qm, jj]],
                            ssems[nb]).wait()
                        pltpu.async_copy(h_hbm.at[sidx_c.at[qn, 0]],
                                         rows_v.at[nb], gsems[nb])

        pltpu.make_async_copy(
            rows_v.at[0], agg_sh.at[didx_c.at[0, 0]], ssems[0]).wait()
        pltpu.make_async_copy(
            rows_v.at[1], agg_sh.at[didx_c.at[0, 0]], ssems[1]).wait()
        plsc.subcore_barrier()

        pltpu.sync_copy(agg_sh.at[pl.ds(roff, rows_seg)],
                        out_hbm.at[c, pl.ds(roff, rows_seg)])

    return agg_k


def _mm_body(x_ref, w_ref, h_ref):
    h_ref[...] = jnp.dot(x_ref[...], w_ref[...], preferred_element_type=_f32)


def _scale_body(h_ref, degs_ref, hs_ref, nd_ref):
    degs = degs_ref[...]
    ds = degs[0, 0] + degs[1, 0]
    dd = degs[0, 1] + degs[1, 1]
    ns = jnp.where(ds > 0, lax.rsqrt(jnp.maximum(ds, 1.0)), 0.0)
    nd = jnp.where(dd > 0, lax.rsqrt(jnp.maximum(dd, 1.0)), 0.0)
    hs_ref[...] = h_ref[...] * ns[:, None]
    nd_ref[...] = nd[:, None]


def _ep_body(parts_ref, nd_ref, b_ref, x_ref, o_ref):
    parts = parts_ref[...].astype(_f32)
    agg = parts[0] + parts[1]
    agg = agg * nd_ref[...] + b_ref[...]
    mean = jnp.mean(agg, axis=1, keepdims=True)
    cen = agg - mean
    var = jnp.mean(cen * cen, axis=1, keepdims=True)
    hn = cen / jnp.sqrt(var + EPS)
    o_ref[...] = jnp.maximum(hn, 0.0) + x_ref[...]


def kernel(x, edge_index, W, b):
    n, d_in = x.shape
    d_out = W.shape[1]
    e = edge_index.shape[1]

    n_pad = ((n + 1 + 2047) // 2048) * 2048
    batch = 128
    ept = -(-e // NW)
    n_batches = -(-(-(-ept // batch)) // 8) * 8
    e_pad = NW * n_batches * batch

    src = edge_index[0].astype(_i32)
    dst = edge_index[1].astype(_i32)
    pad_i = n + jnp.arange(e_pad - e, dtype=_i32) % (n_pad - n)
    pad_s = pad_i
    pad_d = pad_i
    src_p = jnp.concatenate([src, pad_s]).reshape(NW, n_batches, batch)
    dst_p = jnp.concatenate([dst, pad_d]).reshape(NW, n_batches, batch)
    deg_k = _degree_kernel(n_pad, n_batches, batch)
    zeros_h = jnp.zeros((n_pad // NS,), _f32)
    degs = deg_k(src_p, dst_p, zeros_h).reshape(NC, 2, n_pad)

    mblk = 1000
    h_mm = pl.pallas_call(
        _mm_body,
        grid=(n // mblk,),
        in_specs=[
            pl.BlockSpec((mblk, d_in), lambda i: (i, 0)),
            pl.BlockSpec((d_in, d_out), lambda i: (0, 0)),
        ],
        out_specs=pl.BlockSpec((mblk, d_out), lambda i: (i, 0)),
        out_shape=jax.ShapeDtypeStruct((n_pad, d_out), _f32),
    )(x, W)

    blk = 1024
    h_scaled, norm_dst = pl.pallas_call(
        _scale_body,
        grid=(n_pad // blk,),
        in_specs=[
            pl.BlockSpec((blk, d_out), lambda i: (i, 0)),
            pl.BlockSpec((NC, 2, blk), lambda i: (0, 0, i)),
        ],
        out_specs=[
            pl.BlockSpec((blk, d_out), lambda i: (i, 0)),
            pl.BlockSpec((blk, 1), lambda i: (i, 0)),
        ],
        out_shape=[
            jax.ShapeDtypeStruct((n_pad, d_out), _f32),
            jax.ShapeDtypeStruct((n_pad, 1), _f32),
        ],
    )(h_mm, degs)

    agg_k = _agg_kernel(n_pad, n_batches, batch, _f32)
    zrows = jnp.zeros((n_pad // NS, 128), _f32)
    parts = agg_k(h_scaled, src_p, dst_p, zrows)

    eblk = 2000
    out = pl.pallas_call(
        _ep_body,
        grid=(n // eblk,),
        in_specs=[
            pl.BlockSpec((NC, eblk, 128), lambda i: (0, i, 0)),
            pl.BlockSpec((eblk, 1), lambda i: (i, 0)),
            pl.BlockSpec((1, 128), lambda i: (0, 0)),
            pl.BlockSpec((eblk, 128), lambda i: (i, 0)),
        ],
        out_specs=pl.BlockSpec((eblk, 128), lambda i: (i, 0)),
        out_shape=jax.ShapeDtypeStruct((n, d_out), _f32),
    )(parts, norm_dst[:n], b.reshape(1, 128), x)

    return out
